# baseline (speedup 1.0000x reference)
import functools

import jax
import jax.numpy as jnp
from jax import lax
from jax.experimental import pallas as pl
from jax.experimental.pallas import tpu as pltpu
from jax.experimental.pallas import tpu_sc as plsc

NUM_CORES = 2
NUM_SUBCORES = 16
NUM_WORKERS = NUM_CORES * NUM_SUBCORES
LANES = 16

NUM_ROWS = 1000000
BATCH = 16384
DIM = 32
B_PER_W = BATCH // NUM_WORKERS
GROUPS = B_PER_W // LANES
N_ELEMS = B_PER_W * DIM
GCHUNK = 128
WAVE = 8


def _mf_body(uidx_hbm, iidx_hbm, ut_hbm, it_hbm, w_hbm, out_hbm,
             idx_u, idx_i, ids_u, ids_i, g_u, g_i, w_v, out_v, sem):
    wid = lax.axis_index("s") * NUM_CORES + lax.axis_index("c")
    base = wid * B_PER_W

    pltpu.sync_copy(uidx_hbm.at[wid], idx_u)
    pltpu.sync_copy(iidx_hbm.at[wid], idx_i)
    pltpu.sync_copy(w_hbm, w_v)

    def build_body(g, carry):
        u16 = idx_u[pl.ds(g * LANES, LANES)]
        i16 = idx_i[pl.ds(g * LANES, LANES)]
        for d in range(DIM):
            off = jnp.full((LANES,), d * NUM_ROWS, jnp.int32)
            ids_u[pl.ds(d * B_PER_W + g * LANES, LANES)] = u16 + off
            ids_i[pl.ds(d * B_PER_W + g * LANES, LANES)] = i16 + off
        return carry

    lax.fori_loop(0, GROUPS, build_body, 0)

    def gather_wave(jw, carry):
        cps = []
        for t in range(WAVE):
            j = jw * WAVE + t
            sl = pl.ds(j * GCHUNK, GCHUNK)
            cps.append(pltpu.async_copy(
                ut_hbm.at[ids_u.at[sl]], g_u.at[sl], sem))
            cps.append(pltpu.async_copy(
                it_hbm.at[ids_i.at[sl]], g_i.at[sl], sem))
        for c in cps:
            c.wait()
        return carry

    lax.fori_loop(0, N_ELEMS // (GCHUNK * WAVE), gather_wave, 0)

    def group_body(g, carry):
        acc = jnp.zeros((LANES,), jnp.float32)
        for d in range(DIM):
            off = d * B_PER_W + g * LANES
            acc = acc + (g_u[pl.ds(off, LANES)] * g_i[pl.ds(off, LANES)]
                         * w_v[d, :])
        sig = 1.0 / (1.0 + jnp.exp(-acc))
        out_v[pl.ds(g * LANES, LANES)] = sig
        return carry

    lax.fori_loop(0, GROUPS, group_body, 0)

    pltpu.sync_copy(out_v, out_hbm.at[pl.ds(base, B_PER_W)])


@functools.partial(
    pl.kernel,
    out_type=jax.ShapeDtypeStruct((BATCH,), jnp.float32),
    mesh=plsc.VectorSubcoreMesh(core_axis_name="c", subcore_axis_name="s"),
    scratch_types=[
        pltpu.VMEM((B_PER_W,), jnp.int32),
        pltpu.VMEM((B_PER_W,), jnp.int32),
        pltpu.VMEM((N_ELEMS,), jnp.int32),
        pltpu.VMEM((N_ELEMS,), jnp.int32),
        pltpu.VMEM((N_ELEMS,), jnp.float32),
        pltpu.VMEM((N_ELEMS,), jnp.float32),
        pltpu.VMEM((DIM, LANES), jnp.float32),
        pltpu.VMEM((B_PER_W,), jnp.float32),
        pltpu.SemaphoreType.DMA,
    ],
    compiler_params=pltpu.CompilerParams(
        needs_layout_passes=False, use_tc_tiling_on_sc=False),
)
def _mf_kernel(*refs):
    _mf_body(*refs)


def kernel(user_indices, item_indices, user_emb, item_emb, fc_w):
    uidx = user_indices.astype(jnp.int32).reshape(NUM_WORKERS, B_PER_W)
    iidx = item_indices.astype(jnp.int32).reshape(NUM_WORKERS, B_PER_W)
    ut = user_emb.T.reshape(NUM_ROWS * DIM)
    it = item_emb.T.reshape(NUM_ROWS * DIM)
    w_b = jnp.broadcast_to(fc_w.reshape(DIM, 1), (DIM, LANES))
    return _mf_kernel(uidx, iidx, ut, it, w_b)


# speedup vs baseline: 5.5408x; 5.5408x over previous
import functools

import jax
import jax.numpy as jnp
from jax import lax
from jax.experimental import pallas as pl
from jax.experimental.pallas import tpu as pltpu
from jax.experimental.pallas import tpu_sc as plsc

NUM_CORES = 2
NUM_SUBCORES = 16
NUM_WORKERS = NUM_CORES * NUM_SUBCORES
LANES = 16

NUM_ROWS = 1000000
BATCH = 16384
DIM = 32
ROWS_PER_WIDE = 128 // DIM
WIDE_ROWS = NUM_ROWS // ROWS_PER_WIDE
B_PER_W = BATCH // NUM_WORKERS
GROUPS = B_PER_W // LANES
GCHUNK = 128
WAVE = 4


def _mf_body(uidx_hbm, iidx_hbm, ut_hbm, it_hbm, w_hbm, out_hbm,
             idx_u, idx_i, rid, rbuf, g_u, g_i, w_v, out_v, sem):
    wid = lax.axis_index("s") * NUM_CORES + lax.axis_index("c")
    base = wid * B_PER_W

    pltpu.sync_copy(uidx_hbm.at[wid], idx_u)
    pltpu.sync_copy(iidx_hbm.at[wid], idx_i)
    pltpu.sync_copy(w_hbm, w_v)

    iota = lax.iota(jnp.int32, LANES)

    for tab_hbm, idx_t, g_t in ((ut_hbm, idx_u, g_u), (it_hbm, idx_i, g_i)):
        def build_body(g, carry):
            i16 = idx_t[pl.ds(g * LANES, LANES)]
            rid[pl.ds(g * LANES, LANES)] = lax.shift_right_logical(i16, 2)
            return carry

        lax.fori_loop(0, GROUPS, build_body, 0)

        def gather_wave(wv, carry):
            cps = []
            for t in range(WAVE):
                j = wv * WAVE + t
                cps.append(pltpu.async_copy(
                    tab_hbm.at[rid.at[pl.ds(j * GCHUNK, GCHUNK)]],
                    rbuf.at[pl.ds(j * GCHUNK, GCHUNK)], sem))
            for c in cps:
                c.wait()
            return carry

        lax.fori_loop(0, B_PER_W // (GCHUNK * WAVE), gather_wave, 0)

        def extract_body(g, carry):
            i16 = idx_t[pl.ds(g * LANES, LANES)]
            sub = lax.bitwise_and(i16, jnp.full((LANES,), 3, jnp.int32))
            colbase = sub * DIM
            row16 = jnp.full((LANES,), g * LANES, jnp.int32) + iota
            for d in range(DIM):
                val = plsc.load_gather(rbuf, [row16, colbase + d])
                g_t[pl.ds(d * B_PER_W + g * LANES, LANES)] = val
            return carry

        lax.fori_loop(0, GROUPS, extract_body, 0)

    def group_body(g, carry):
        acc = jnp.zeros((LANES,), jnp.float32)
        for d in range(DIM):
            off = d * B_PER_W + g * LANES
            acc = acc + (g_u[pl.ds(off, LANES)] * g_i[pl.ds(off, LANES)]
                         * w_v[d, :])
        sig = 1.0 / (1.0 + jnp.exp(-acc))
        out_v[pl.ds(g * LANES, LANES)] = sig
        return carry

    lax.fori_loop(0, GROUPS, group_body, 0)

    pltpu.sync_copy(out_v, out_hbm.at[pl.ds(base, B_PER_W)])


@functools.partial(
    pl.kernel,
    out_type=jax.ShapeDtypeStruct((BATCH,), jnp.float32),
    mesh=plsc.VectorSubcoreMesh(core_axis_name="c", subcore_axis_name="s"),
    scratch_types=[
        pltpu.VMEM((B_PER_W,), jnp.int32),
        pltpu.VMEM((B_PER_W,), jnp.int32),
        pltpu.VMEM((B_PER_W,), jnp.int32),
        pltpu.VMEM((B_PER_W, 128), jnp.float32),
        pltpu.VMEM((B_PER_W * DIM,), jnp.float32),
        pltpu.VMEM((B_PER_W * DIM,), jnp.float32),
        pltpu.VMEM((DIM, LANES), jnp.float32),
        pltpu.VMEM((B_PER_W,), jnp.float32),
        pltpu.SemaphoreType.DMA,
    ],
    compiler_params=pltpu.CompilerParams(
        needs_layout_passes=False, use_tc_tiling_on_sc=True),
)
def _mf_kernel(*refs):
    _mf_body(*refs)


def kernel(user_indices, item_indices, user_emb, item_emb, fc_w):
    uidx = user_indices.astype(jnp.int32).reshape(NUM_WORKERS, B_PER_W)
    iidx = item_indices.astype(jnp.int32).reshape(NUM_WORKERS, B_PER_W)
    ut = user_emb.reshape(WIDE_ROWS, 128)
    it = item_emb.reshape(WIDE_ROWS, 128)
    w_b = jnp.broadcast_to(fc_w.reshape(DIM, 1), (DIM, LANES))
    return _mf_kernel(uidx, iidx, ut, it, w_b)


# speedup vs baseline: 14.2257x; 2.5674x over previous
import functools

import jax
import jax.numpy as jnp
from jax import lax
from jax.experimental import pallas as pl
from jax.experimental.pallas import tpu as pltpu
from jax.experimental.pallas import tpu_sc as plsc

NUM_CORES = 2
NUM_SUBCORES = 16
NUM_WORKERS = NUM_CORES * NUM_SUBCORES
LANES = 16

NUM_ROWS = 1000000
BATCH = 16384
DIM = 32
B_PER_W = BATCH // NUM_WORKERS
GROUPS = B_PER_W // LANES

TILE_W = 128
FULL_TILES = NUM_ROWS // TILE_W
ALIGNED = FULL_TILES * TILE_W
TAIL = NUM_ROWS - ALIGNED
DBLKS = DIM // 8
PACK_ROWS = DBLKS * FULL_TILES
TILES_PER_W = 256
LAST_START = FULL_TILES - TILES_PER_W
WAVE_T = 16
WAVES = TILES_PER_W // WAVE_T


def _lin_body(ut_hbm, it_hbm, lu_hbm, li_hbm, tbuf, sem):
    wid = lax.axis_index("s") * NUM_CORES + lax.axis_index("c")
    start_tile = jnp.minimum(wid * TILES_PER_W, LAST_START)

    for tab_hbm, lin_hbm in ((ut_hbm, lu_hbm), (it_hbm, li_hbm)):
        for dblk in range(DBLKS):

            def wave_body(k, carry):
                cps = []
                for j in range(WAVE_T):
                    t = start_tile + k * WAVE_T + j
                    i0 = pl.multiple_of(t * TILE_W, TILE_W)
                    cps.append(pltpu.async_copy(
                        tab_hbm.at[pl.ds(dblk * 8, 8), pl.ds(i0, TILE_W)],
                        tbuf.at[j], sem))
                for c in cps:
                    c.wait()
                cps = []
                for j in range(WAVE_T):
                    t = start_tile + k * WAVE_T + j
                    cps.append(pltpu.async_copy(
                        tbuf.at[j], lin_hbm.at[dblk * FULL_TILES + t], sem))
                for c in cps:
                    c.wait()
                return carry

            lax.fori_loop(0, WAVES, wave_body, 0)


@functools.partial(
    pl.kernel,
    out_type=(jax.ShapeDtypeStruct((PACK_ROWS, 8, TILE_W), jnp.float32),
              jax.ShapeDtypeStruct((PACK_ROWS, 8, TILE_W), jnp.float32)),
    mesh=plsc.VectorSubcoreMesh(core_axis_name="c", subcore_axis_name="s"),
    scratch_types=[
        pltpu.VMEM((WAVE_T, 8, TILE_W), jnp.float32),
        pltpu.SemaphoreType.DMA,
    ],
    compiler_params=pltpu.CompilerParams(
        needs_layout_passes=False, use_tc_tiling_on_sc=True),
)
def _lin_kernel(*refs):
    _lin_body(*refs)


def _mf_body(uidx_hbm, iidx_hbm, lu_hbm, li_hbm, tu_hbm, ti_hbm, w_hbm,
             out_hbm, idx_u, idx_i, ids_u, ids_i, g_u, g_i, tl_u, tl_i,
             w_v, out_v, sem):
    wid = lax.axis_index("s") * NUM_CORES + lax.axis_index("c")
    base = wid * B_PER_W

    pltpu.sync_copy(uidx_hbm.at[wid], idx_u)
    pltpu.sync_copy(iidx_hbm.at[wid], idx_i)
    pltpu.sync_copy(w_hbm, w_v)
    pltpu.sync_copy(tu_hbm, tl_u)
    pltpu.sync_copy(ti_hbm, tl_i)

    clamp = jnp.full((LANES,), ALIGNED - 1, jnp.int32)
    m127 = jnp.full((LANES,), TILE_W - 1, jnp.int32)

    def build_body(g, carry):
        u16 = jnp.minimum(idx_u[pl.ds(g * LANES, LANES)], clamp)
        i16 = jnp.minimum(idx_i[pl.ds(g * LANES, LANES)], clamp)
        su = lax.shift_left(lax.shift_right_logical(u16, 7), 10) \
            + lax.bitwise_and(u16, m127)
        si = lax.shift_left(lax.shift_right_logical(i16, 7), 10) \
            + lax.bitwise_and(i16, m127)
        for d in range(DIM):
            cd = (d // 8) * FULL_TILES * 1024 + (d % 8) * TILE_W
            off = jnp.full((LANES,), cd, jnp.int32)
            ids_u[pl.ds(d * B_PER_W + g * LANES, LANES)] = su + off
            ids_i[pl.ds(d * B_PER_W + g * LANES, LANES)] = si + off
        return carry

    lax.fori_loop(0, GROUPS, build_body, 0)

    cu = pltpu.async_copy(lu_hbm.at[ids_u], g_u, sem)
    ci = pltpu.async_copy(li_hbm.at[ids_i], g_i, sem)
    cu.wait()
    ci.wait()

    tail_lo = jnp.full((LANES,), ALIGNED, jnp.int32)
    zero16 = jnp.zeros((LANES,), jnp.int32)

    def group_body(g, carry):
        iu16 = idx_u[pl.ds(g * LANES, LANES)]
        ii16 = idx_i[pl.ds(g * LANES, LANES)]
        mu = iu16 >= tail_lo
        mi = ii16 >= tail_lo
        tu16 = jnp.maximum(iu16 - tail_lo, zero16)
        ti16 = jnp.maximum(ii16 - tail_lo, zero16)
        acc = jnp.zeros((LANES,), jnp.float32)
        for d in range(DIM):
            off = d * B_PER_W + g * LANES
            d16 = jnp.full((LANES,), d, jnp.int32)
            uval = jnp.where(mu, plsc.load_gather(tl_u, [tu16, d16]),
                             g_u[pl.ds(off, LANES)])
            ival = jnp.where(mi, plsc.load_gather(tl_i, [ti16, d16]),
                             g_i[pl.ds(off, LANES)])
            acc = acc + uval * ival * w_v[d, :]
        sig = 1.0 / (1.0 + jnp.exp(-acc))
        out_v[pl.ds(g * LANES, LANES)] = sig
        return carry

    lax.fori_loop(0, GROUPS, group_body, 0)

    pltpu.sync_copy(out_v, out_hbm.at[pl.ds(base, B_PER_W)])


@functools.partial(
    pl.kernel,
    out_type=jax.ShapeDtypeStruct((BATCH,), jnp.float32),
    mesh=plsc.VectorSubcoreMesh(core_axis_name="c", subcore_axis_name="s"),
    scratch_types=[
        pltpu.VMEM((B_PER_W,), jnp.int32),
        pltpu.VMEM((B_PER_W,), jnp.int32),
        pltpu.VMEM((B_PER_W * DIM,), jnp.int32),
        pltpu.VMEM((B_PER_W * DIM,), jnp.int32),
        pltpu.VMEM((B_PER_W * DIM,), jnp.float32),
        pltpu.VMEM((B_PER_W * DIM,), jnp.float32),
        pltpu.VMEM((TAIL, DIM), jnp.float32),
        pltpu.VMEM((TAIL, DIM), jnp.float32),
        pltpu.VMEM((DIM, LANES), jnp.float32),
        pltpu.VMEM((B_PER_W,), jnp.float32),
        pltpu.SemaphoreType.DMA,
    ],
    compiler_params=pltpu.CompilerParams(
        needs_layout_passes=False, use_tc_tiling_on_sc=False),
)
def _mf_kernel(*refs):
    _mf_body(*refs)


def kernel(user_indices, item_indices, user_emb, item_emb, fc_w):
    uidx = user_indices.astype(jnp.int32).reshape(NUM_WORKERS, B_PER_W)
    iidx = item_indices.astype(jnp.int32).reshape(NUM_WORKERS, B_PER_W)
    ut = user_emb.T
    it = item_emb.T
    tail_u = user_emb[ALIGNED:, :]
    tail_i = item_emb[ALIGNED:, :]
    w_b = jnp.broadcast_to(fc_w.reshape(DIM, 1), (DIM, LANES))
    lin_u, lin_i = _lin_kernel(ut, it)
    lin_u = lin_u.reshape(PACK_ROWS * 8 * TILE_W)
    lin_i = lin_i.reshape(PACK_ROWS * 8 * TILE_W)
    return _mf_kernel(uidx, iidx, lin_u, lin_i, tail_u, tail_i, w_b)


# speedup vs baseline: 14.4305x; 1.0144x over previous
import functools

import jax
import jax.numpy as jnp
from jax import lax
from jax.experimental import pallas as pl
from jax.experimental.pallas import tpu as pltpu
from jax.experimental.pallas import tpu_sc as plsc

NUM_CORES = 2
NUM_SUBCORES = 16
NUM_WORKERS = NUM_CORES * NUM_SUBCORES
LANES = 16

NUM_ROWS = 1000000
BATCH = 16384
DIM = 32
B_PER_W = BATCH // NUM_WORKERS
GROUPS = B_PER_W // LANES

TILE_W = 128
FULL_TILES = NUM_ROWS // TILE_W
ALIGNED = FULL_TILES * TILE_W
TAIL = NUM_ROWS - ALIGNED
DBLKS = DIM // 8
PACK_ROWS = DBLKS * FULL_TILES
TILES_PER_W = 256
LAST_START = FULL_TILES - TILES_PER_W
WAVE_T = 16
WAVES = TILES_PER_W // WAVE_T


def _lin_body(ut_hbm, it_hbm, lu_hbm, li_hbm, tbuf_a, tbuf_b, sem_r, sem_w):
    wid = lax.axis_index("s") * NUM_CORES + lax.axis_index("c")
    start_tile = jnp.minimum(wid * TILES_PER_W, LAST_START)

    for tab_hbm, lin_hbm in ((ut_hbm, lu_hbm), (it_hbm, li_hbm)):
        for dblk in range(DBLKS):

            def read_wave(k, buf):
                cps = []
                for j in range(WAVE_T):
                    t = start_tile + k * WAVE_T + j
                    i0 = pl.multiple_of(t * TILE_W, TILE_W)
                    cps.append(pltpu.async_copy(
                        tab_hbm.at[pl.ds(dblk * 8, 8), pl.ds(i0, TILE_W)],
                        buf.at[j], sem_r))
                for c in cps:
                    c.wait()

            def write_wave(k, buf):
                cps = []
                for j in range(WAVE_T):
                    t = start_tile + k * WAVE_T + j
                    cps.append(pltpu.async_copy(
                        buf.at[j], lin_hbm.at[dblk * FULL_TILES + t], sem_w))
                return cps

            def drain_writes():
                for j in range(WAVE_T):
                    pltpu.make_async_copy(
                        lin_hbm.at[0], tbuf_b.at[j], sem_w).wait()

            def wave_pair(p, carry):
                @pl.when(p > 0)
                def _():
                    drain_writes()
                read_wave(2 * p, tbuf_a)
                wa = write_wave(2 * p, tbuf_a)
                read_wave(2 * p + 1, tbuf_b)
                for c in wa:
                    c.wait()
                write_wave(2 * p + 1, tbuf_b)
                return carry

            lax.fori_loop(0, WAVES // 2, wave_pair, 0)
            drain_writes()


@functools.partial(
    pl.kernel,
    out_type=(jax.ShapeDtypeStruct((PACK_ROWS, 8, TILE_W), jnp.float32),
              jax.ShapeDtypeStruct((PACK_ROWS, 8, TILE_W), jnp.float32)),
    mesh=plsc.VectorSubcoreMesh(core_axis_name="c", subcore_axis_name="s"),
    scratch_types=[
        pltpu.VMEM((WAVE_T, 8, TILE_W), jnp.float32),
        pltpu.VMEM((WAVE_T, 8, TILE_W), jnp.float32),
        pltpu.SemaphoreType.DMA,
        pltpu.SemaphoreType.DMA,
    ],
    compiler_params=pltpu.CompilerParams(
        needs_layout_passes=False, use_tc_tiling_on_sc=True),
)
def _lin_kernel(*refs):
    _lin_body(*refs)


def _mf_body(uidx_hbm, iidx_hbm, lu_hbm, li_hbm, tu_hbm, ti_hbm, w_hbm,
             out_hbm, idx_u, idx_i, ids_u, ids_i, g_u, g_i, tl_u, tl_i,
             w_v, out_v, sem):
    wid = lax.axis_index("s") * NUM_CORES + lax.axis_index("c")
    base = wid * B_PER_W

    pltpu.sync_copy(uidx_hbm.at[wid], idx_u)
    pltpu.sync_copy(iidx_hbm.at[wid], idx_i)
    pltpu.sync_copy(w_hbm, w_v)
    pltpu.sync_copy(tu_hbm, tl_u)
    pltpu.sync_copy(ti_hbm, tl_i)

    clamp = jnp.full((LANES,), ALIGNED - 1, jnp.int32)
    m127 = jnp.full((LANES,), TILE_W - 1, jnp.int32)

    def build_body(g, carry):
        u16 = jnp.minimum(idx_u[pl.ds(g * LANES, LANES)], clamp)
        i16 = jnp.minimum(idx_i[pl.ds(g * LANES, LANES)], clamp)
        su = lax.shift_left(lax.shift_right_logical(u16, 7), 10) \
            + lax.bitwise_and(u16, m127)
        si = lax.shift_left(lax.shift_right_logical(i16, 7), 10) \
            + lax.bitwise_and(i16, m127)
        for d in range(DIM):
            cd = (d // 8) * FULL_TILES * 1024 + (d % 8) * TILE_W
            off = jnp.full((LANES,), cd, jnp.int32)
            ids_u[pl.ds(d * B_PER_W + g * LANES, LANES)] = su + off
            ids_i[pl.ds(d * B_PER_W + g * LANES, LANES)] = si + off
        return carry

    lax.fori_loop(0, GROUPS, build_body, 0)

    cu = pltpu.async_copy(lu_hbm.at[ids_u], g_u, sem)
    ci = pltpu.async_copy(li_hbm.at[ids_i], g_i, sem)
    cu.wait()
    ci.wait()

    tail_lo = jnp.full((LANES,), ALIGNED, jnp.int32)
    zero16 = jnp.zeros((LANES,), jnp.int32)

    def group_body(g, carry):
        iu16 = idx_u[pl.ds(g * LANES, LANES)]
        ii16 = idx_i[pl.ds(g * LANES, LANES)]
        mu = iu16 >= tail_lo
        mi = ii16 >= tail_lo
        tu16 = jnp.maximum(iu16 - tail_lo, zero16)
        ti16 = jnp.maximum(ii16 - tail_lo, zero16)
        acc = jnp.zeros((LANES,), jnp.float32)
        for d in range(DIM):
            off = d * B_PER_W + g * LANES
            d16 = jnp.full((LANES,), d, jnp.int32)
            uval = jnp.where(mu, plsc.load_gather(tl_u, [tu16, d16]),
                             g_u[pl.ds(off, LANES)])
            ival = jnp.where(mi, plsc.load_gather(tl_i, [ti16, d16]),
                             g_i[pl.ds(off, LANES)])
            acc = acc + uval * ival * w_v[d, :]
        sig = 1.0 / (1.0 + jnp.exp(-acc))
        out_v[pl.ds(g * LANES, LANES)] = sig
        return carry

    lax.fori_loop(0, GROUPS, group_body, 0)

    pltpu.sync_copy(out_v, out_hbm.at[pl.ds(base, B_PER_W)])


@functools.partial(
    pl.kernel,
    out_type=jax.ShapeDtypeStruct((BATCH,), jnp.float32),
    mesh=plsc.VectorSubcoreMesh(core_axis_name="c", subcore_axis_name="s"),
    scratch_types=[
        pltpu.VMEM((B_PER_W,), jnp.int32),
        pltpu.VMEM((B_PER_W,), jnp.int32),
        pltpu.VMEM((B_PER_W * DIM,), jnp.int32),
        pltpu.VMEM((B_PER_W * DIM,), jnp.int32),
        pltpu.VMEM((B_PER_W * DIM,), jnp.float32),
        pltpu.VMEM((B_PER_W * DIM,), jnp.float32),
        pltpu.VMEM((TAIL, DIM), jnp.float32),
        pltpu.VMEM((TAIL, DIM), jnp.float32),
        pltpu.VMEM((DIM, LANES), jnp.float32),
        pltpu.VMEM((B_PER_W,), jnp.float32),
        pltpu.SemaphoreType.DMA,
    ],
    compiler_params=pltpu.CompilerParams(
        needs_layout_passes=False, use_tc_tiling_on_sc=False),
)
def _mf_kernel(*refs):
    _mf_body(*refs)


def kernel(user_indices, item_indices, user_emb, item_emb, fc_w):
    uidx = user_indices.astype(jnp.int32).reshape(NUM_WORKERS, B_PER_W)
    iidx = item_indices.astype(jnp.int32).reshape(NUM_WORKERS, B_PER_W)
    ut = user_emb.T
    it = item_emb.T
    tail_u = user_emb[ALIGNED:, :]
    tail_i = item_emb[ALIGNED:, :]
    w_b = jnp.broadcast_to(fc_w.reshape(DIM, 1), (DIM, LANES))
    lin_u, lin_i = _lin_kernel(ut, it)
    lin_u = lin_u.reshape(PACK_ROWS * 8 * TILE_W)
    lin_i = lin_i.reshape(PACK_ROWS * 8 * TILE_W)
    return _mf_kernel(uidx, iidx, lin_u, lin_i, tail_u, tail_i, w_b)


# speedup vs baseline: 15.9399x; 1.1046x over previous
import functools

import jax
import jax.numpy as jnp
from jax import lax
from jax.experimental import pallas as pl
from jax.experimental.pallas import tpu as pltpu
from jax.experimental.pallas import tpu_sc as plsc

NUM_CORES = 2
NUM_SUBCORES = 16
NUM_WORKERS = NUM_CORES * NUM_SUBCORES
LANES = 16

NUM_ROWS = 1000000
BATCH = 16384
DIM = 32
B_PER_W = BATCH // NUM_WORKERS
GROUPS = B_PER_W // LANES

TILE_W = 128
FULL_TILES = NUM_ROWS // TILE_W
ALIGNED = FULL_TILES * TILE_W
TAIL = NUM_ROWS - ALIGNED
DBLKS = DIM // 8
PACK_ROWS = DBLKS * FULL_TILES
TILES_PER_W = 256
LAST_START = FULL_TILES - TILES_PER_W
WAVE_T = 16
WAVES = TILES_PER_W // WAVE_T


def _lin_body(ut_hbm, it_hbm, lu_hbm, li_hbm, tbuf_a, tbuf_b, sem_r, sem_w):
    wid = lax.axis_index("s") * NUM_CORES + lax.axis_index("c")
    start_tile = jnp.minimum(wid * TILES_PER_W, LAST_START)

    for tab_hbm, lin_hbm in ((ut_hbm, lu_hbm), (it_hbm, li_hbm)):
        for dblk in range(DBLKS):

            def fire_reads(k, buf):
                cps = []
                for j in range(WAVE_T):
                    t = start_tile + k * WAVE_T + j
                    i0 = pl.multiple_of(t * TILE_W, TILE_W)
                    cps.append(pltpu.async_copy(
                        tab_hbm.at[pl.ds(dblk * 8, 8), pl.ds(i0, TILE_W)],
                        buf.at[j], sem_r))
                return cps

            def write_wave(k, buf):
                cps = []
                for j in range(WAVE_T):
                    t = start_tile + k * WAVE_T + j
                    cps.append(pltpu.async_copy(
                        buf.at[j], lin_hbm.at[dblk * FULL_TILES + t], sem_w))
                return cps

            def drain_writes():
                for j in range(WAVE_T):
                    pltpu.make_async_copy(
                        lin_hbm.at[0], tbuf_b.at[j], sem_w).wait()

            def wave_pair(p, carry):
                ra = fire_reads(2 * p, tbuf_a)

                @pl.when(p > 0)
                def _():
                    drain_writes()
                for c in ra:
                    c.wait()
                wa = write_wave(2 * p, tbuf_a)
                rb = fire_reads(2 * p + 1, tbuf_b)
                for c in rb:
                    c.wait()
                for c in wa:
                    c.wait()
                write_wave(2 * p + 1, tbuf_b)
                return carry

            lax.fori_loop(0, WAVES // 2, wave_pair, 0)
            drain_writes()


@functools.partial(
    pl.kernel,
    out_type=(jax.ShapeDtypeStruct((PACK_ROWS, 8, TILE_W), jnp.float32),
              jax.ShapeDtypeStruct((PACK_ROWS, 8, TILE_W), jnp.float32)),
    mesh=plsc.VectorSubcoreMesh(core_axis_name="c", subcore_axis_name="s"),
    scratch_types=[
        pltpu.VMEM((WAVE_T, 8, TILE_W), jnp.float32),
        pltpu.VMEM((WAVE_T, 8, TILE_W), jnp.float32),
        pltpu.SemaphoreType.DMA,
        pltpu.SemaphoreType.DMA,
    ],
    compiler_params=pltpu.CompilerParams(
        needs_layout_passes=False, use_tc_tiling_on_sc=True),
)
def _lin_kernel(*refs):
    _lin_body(*refs)


def _mf_body(uidx_hbm, iidx_hbm, lu_hbm, li_hbm, tu_hbm, ti_hbm, w_hbm,
             out_hbm, idx_u, idx_i, ids_u, ids_i, g_u, g_i, tl_u, tl_i,
             w_v, out_v, sem):
    wid = lax.axis_index("s") * NUM_CORES + lax.axis_index("c")
    base = wid * B_PER_W

    pltpu.sync_copy(uidx_hbm.at[wid], idx_u)
    pltpu.sync_copy(iidx_hbm.at[wid], idx_i)
    pltpu.sync_copy(w_hbm, w_v)
    pltpu.sync_copy(tu_hbm, tl_u)
    pltpu.sync_copy(ti_hbm, tl_i)

    clamp = jnp.full((LANES,), ALIGNED - 1, jnp.int32)
    m127 = jnp.full((LANES,), TILE_W - 1, jnp.int32)

    def build_body(g, carry):
        u16 = jnp.minimum(idx_u[pl.ds(g * LANES, LANES)], clamp)
        i16 = jnp.minimum(idx_i[pl.ds(g * LANES, LANES)], clamp)
        su = lax.shift_left(lax.shift_right_logical(u16, 7), 10) \
            + lax.bitwise_and(u16, m127)
        si = lax.shift_left(lax.shift_right_logical(i16, 7), 10) \
            + lax.bitwise_and(i16, m127)
        for d in range(DIM):
            cd = (d // 8) * FULL_TILES * 1024 + (d % 8) * TILE_W
            off = jnp.full((LANES,), cd, jnp.int32)
            ids_u[pl.ds(d * B_PER_W + g * LANES, LANES)] = su + off
            ids_i[pl.ds(d * B_PER_W + g * LANES, LANES)] = si + off
        return carry

    lax.fori_loop(0, GROUPS, build_body, 0)

    cu = pltpu.async_copy(lu_hbm.at[ids_u], g_u, sem)
    ci = pltpu.async_copy(li_hbm.at[ids_i], g_i, sem)
    cu.wait()
    ci.wait()

    tail_lo = jnp.full((LANES,), ALIGNED, jnp.int32)
    zero16 = jnp.zeros((LANES,), jnp.int32)

    def group_body(g, carry):
        iu16 = idx_u[pl.ds(g * LANES, LANES)]
        ii16 = idx_i[pl.ds(g * LANES, LANES)]
        mu = iu16 >= tail_lo
        mi = ii16 >= tail_lo
        tu16 = jnp.maximum(iu16 - tail_lo, zero16)
        ti16 = jnp.maximum(ii16 - tail_lo, zero16)
        acc = jnp.zeros((LANES,), jnp.float32)
        for d in range(DIM):
            off = d * B_PER_W + g * LANES
            d16 = jnp.full((LANES,), d, jnp.int32)
            uval = jnp.where(mu, plsc.load_gather(tl_u, [tu16, d16]),
                             g_u[pl.ds(off, LANES)])
            ival = jnp.where(mi, plsc.load_gather(tl_i, [ti16, d16]),
                             g_i[pl.ds(off, LANES)])
            acc = acc + uval * ival * w_v[d, :]
        sig = 1.0 / (1.0 + jnp.exp(-acc))
        out_v[pl.ds(g * LANES, LANES)] = sig
        return carry

    lax.fori_loop(0, GROUPS, group_body, 0)

    pltpu.sync_copy(out_v, out_hbm.at[pl.ds(base, B_PER_W)])


@functools.partial(
    pl.kernel,
    out_type=jax.ShapeDtypeStruct((BATCH,), jnp.float32),
    mesh=plsc.VectorSubcoreMesh(core_axis_name="c", subcore_axis_name="s"),
    scratch_types=[
        pltpu.VMEM((B_PER_W,), jnp.int32),
        pltpu.VMEM((B_PER_W,), jnp.int32),
        pltpu.VMEM((B_PER_W * DIM,), jnp.int32),
        pltpu.VMEM((B_PER_W * DIM,), jnp.int32),
        pltpu.VMEM((B_PER_W * DIM,), jnp.float32),
        pltpu.VMEM((B_PER_W * DIM,), jnp.float32),
        pltpu.VMEM((TAIL, DIM), jnp.float32),
        pltpu.VMEM((TAIL, DIM), jnp.float32),
        pltpu.VMEM((DIM, LANES), jnp.float32),
        pltpu.VMEM((B_PER_W,), jnp.float32),
        pltpu.SemaphoreType.DMA,
    ],
    compiler_params=pltpu.CompilerParams(
        needs_layout_passes=False, use_tc_tiling_on_sc=False),
)
def _mf_kernel(*refs):
    _mf_body(*refs)


def kernel(user_indices, item_indices, user_emb, item_emb, fc_w):
    uidx = user_indices.astype(jnp.int32).reshape(NUM_WORKERS, B_PER_W)
    iidx = item_indices.astype(jnp.int32).reshape(NUM_WORKERS, B_PER_W)
    ut = user_emb.T
    it = item_emb.T
    tail_u = user_emb[ALIGNED:, :]
    tail_i = item_emb[ALIGNED:, :]
    w_b = jnp.broadcast_to(fc_w.reshape(DIM, 1), (DIM, LANES))
    lin_u, lin_i = _lin_kernel(ut, it)
    lin_u = lin_u.reshape(PACK_ROWS * 8 * TILE_W)
    lin_i = lin_i.reshape(PACK_ROWS * 8 * TILE_W)
    return _mf_kernel(uidx, iidx, lin_u, lin_i, tail_u, tail_i, w_b)


# speedup vs baseline: 17.8957x; 1.1227x over previous
import functools

import jax
import jax.numpy as jnp
from jax import lax
from jax.experimental import pallas as pl
from jax.experimental.pallas import tpu as pltpu
from jax.experimental.pallas import tpu_sc as plsc

NUM_CORES = 2
NUM_SUBCORES = 16
NUM_WORKERS = NUM_CORES * NUM_SUBCORES
LANES = 16

NUM_ROWS = 1000000
BATCH = 16384
DIM = 32
B_PER_W = BATCH // NUM_WORKERS
GROUPS = B_PER_W // LANES

TILE_W = 128
FULL_TILES = NUM_ROWS // TILE_W
ALIGNED = FULL_TILES * TILE_W
TAIL = NUM_ROWS - ALIGNED
DBLKS = DIM // 8
PACK_ROWS = DBLKS * FULL_TILES
TILES_PER_W = 256
LAST_START = FULL_TILES - TILES_PER_W
WAVE_T = 32
WAVES = TILES_PER_W // WAVE_T


def _lin_body(ut_hbm, it_hbm, lu_hbm, li_hbm, tbuf_a, tbuf_b, sem_r, sem_w):
    wid = lax.axis_index("s") * NUM_CORES + lax.axis_index("c")
    start_tile = jnp.minimum(wid * TILES_PER_W, LAST_START)

    for tab_hbm, lin_hbm in ((ut_hbm, lu_hbm), (it_hbm, li_hbm)):
        for dblk in range(DBLKS):

            def fire_reads(k, buf):
                cps = []
                for j in range(WAVE_T):
                    t = start_tile + k * WAVE_T + j
                    i0 = pl.multiple_of(t * TILE_W, TILE_W)
                    cps.append(pltpu.async_copy(
                        tab_hbm.at[pl.ds(dblk * 8, 8), pl.ds(i0, TILE_W)],
                        buf.at[j], sem_r))
                return cps

            def write_wave(k, buf):
                cps = []
                for j in range(WAVE_T):
                    t = start_tile + k * WAVE_T + j
                    cps.append(pltpu.async_copy(
                        buf.at[j], lin_hbm.at[dblk * FULL_TILES + t], sem_w))
                return cps

            def drain_writes():
                for j in range(WAVE_T):
                    pltpu.make_async_copy(
                        lin_hbm.at[0], tbuf_b.at[j], sem_w).wait()

            def wave_pair(p, carry):
                ra = fire_reads(2 * p, tbuf_a)

                @pl.when(p > 0)
                def _():
                    drain_writes()
                for c in ra:
                    c.wait()
                wa = write_wave(2 * p, tbuf_a)
                rb = fire_reads(2 * p + 1, tbuf_b)
                for c in rb:
                    c.wait()
                for c in wa:
                    c.wait()
                write_wave(2 * p + 1, tbuf_b)
                return carry

            lax.fori_loop(0, WAVES // 2, wave_pair, 0)
            drain_writes()


@functools.partial(
    pl.kernel,
    out_type=(jax.ShapeDtypeStruct((PACK_ROWS, 8, TILE_W), jnp.float32),
              jax.ShapeDtypeStruct((PACK_ROWS, 8, TILE_W), jnp.float32)),
    mesh=plsc.VectorSubcoreMesh(core_axis_name="c", subcore_axis_name="s"),
    scratch_types=[
        pltpu.VMEM((WAVE_T, 8, TILE_W), jnp.float32),
        pltpu.VMEM((WAVE_T, 8, TILE_W), jnp.float32),
        pltpu.SemaphoreType.DMA,
        pltpu.SemaphoreType.DMA,
    ],
    compiler_params=pltpu.CompilerParams(
        needs_layout_passes=False, use_tc_tiling_on_sc=True),
)
def _lin_kernel(*refs):
    _lin_body(*refs)


def _mf_body(uidx_hbm, iidx_hbm, lu_hbm, li_hbm, tu_hbm, ti_hbm, w_hbm,
             out_hbm, idx_u, idx_i, ids_u, ids_i, g_u, g_i, tl_u, tl_i,
             w_v, out_v, sem):
    wid = lax.axis_index("s") * NUM_CORES + lax.axis_index("c")
    base = wid * B_PER_W

    pltpu.sync_copy(uidx_hbm.at[wid], idx_u)
    pltpu.sync_copy(iidx_hbm.at[wid], idx_i)
    pltpu.sync_copy(w_hbm, w_v)
    pltpu.sync_copy(tu_hbm, tl_u)
    pltpu.sync_copy(ti_hbm, tl_i)

    clamp = jnp.full((LANES,), ALIGNED - 1, jnp.int32)
    m127 = jnp.full((LANES,), TILE_W - 1, jnp.int32)

    def build_body(g, carry):
        u16 = jnp.minimum(idx_u[pl.ds(g * LANES, LANES)], clamp)
        i16 = jnp.minimum(idx_i[pl.ds(g * LANES, LANES)], clamp)
        su = lax.shift_left(lax.shift_right_logical(u16, 7), 10) \
            + lax.bitwise_and(u16, m127)
        si = lax.shift_left(lax.shift_right_logical(i16, 7), 10) \
            + lax.bitwise_and(i16, m127)
        for d in range(DIM):
            cd = (d // 8) * FULL_TILES * 1024 + (d % 8) * TILE_W
            off = jnp.full((LANES,), cd, jnp.int32)
            ids_u[pl.ds(d * B_PER_W + g * LANES, LANES)] = su + off
            ids_i[pl.ds(d * B_PER_W + g * LANES, LANES)] = si + off
        return carry

    lax.fori_loop(0, GROUPS, build_body, 0)

    cu = pltpu.async_copy(lu_hbm.at[ids_u], g_u, sem)
    ci = pltpu.async_copy(li_hbm.at[ids_i], g_i, sem)
    cu.wait()
    ci.wait()

    tail_lo = jnp.full((LANES,), ALIGNED, jnp.int32)
    zero16 = jnp.zeros((LANES,), jnp.int32)

    def group_body(g, carry):
        iu16 = idx_u[pl.ds(g * LANES, LANES)]
        ii16 = idx_i[pl.ds(g * LANES, LANES)]
        mu = iu16 >= tail_lo
        mi = ii16 >= tail_lo
        tu16 = jnp.maximum(iu16 - tail_lo, zero16)
        ti16 = jnp.maximum(ii16 - tail_lo, zero16)
        acc = jnp.zeros((LANES,), jnp.float32)
        for d in range(DIM):
            off = d * B_PER_W + g * LANES
            d16 = jnp.full((LANES,), d, jnp.int32)
            uval = jnp.where(mu, plsc.load_gather(tl_u, [tu16, d16]),
                             g_u[pl.ds(off, LANES)])
            ival = jnp.where(mi, plsc.load_gather(tl_i, [ti16, d16]),
                             g_i[pl.ds(off, LANES)])
            acc = acc + uval * ival * w_v[d, :]
        sig = 1.0 / (1.0 + jnp.exp(-acc))
        out_v[pl.ds(g * LANES, LANES)] = sig
        return carry

    lax.fori_loop(0, GROUPS, group_body, 0)

    pltpu.sync_copy(out_v, out_hbm.at[pl.ds(base, B_PER_W)])


@functools.partial(
    pl.kernel,
    out_type=jax.ShapeDtypeStruct((BATCH,), jnp.float32),
    mesh=plsc.VectorSubcoreMesh(core_axis_name="c", subcore_axis_name="s"),
    scratch_types=[
        pltpu.VMEM((B_PER_W,), jnp.int32),
        pltpu.VMEM((B_PER_W,), jnp.int32),
        pltpu.VMEM((B_PER_W * DIM,), jnp.int32),
        pltpu.VMEM((B_PER_W * DIM,), jnp.int32),
        pltpu.VMEM((B_PER_W * DIM,), jnp.float32),
        pltpu.VMEM((B_PER_W * DIM,), jnp.float32),
        pltpu.VMEM((TAIL, DIM), jnp.float32),
        pltpu.VMEM((TAIL, DIM), jnp.float32),
        pltpu.VMEM((DIM, LANES), jnp.float32),
        pltpu.VMEM((B_PER_W,), jnp.float32),
        pltpu.SemaphoreType.DMA,
    ],
    compiler_params=pltpu.CompilerParams(
        needs_layout_passes=False, use_tc_tiling_on_sc=False),
)
def _mf_kernel(*refs):
    _mf_body(*refs)


def kernel(user_indices, item_indices, user_emb, item_emb, fc_w):
    uidx = user_indices.astype(jnp.int32).reshape(NUM_WORKERS, B_PER_W)
    iidx = item_indices.astype(jnp.int32).reshape(NUM_WORKERS, B_PER_W)
    ut = user_emb.T
    it = item_emb.T
    tail_u = user_emb[ALIGNED:, :]
    tail_i = item_emb[ALIGNED:, :]
    w_b = jnp.broadcast_to(fc_w.reshape(DIM, 1), (DIM, LANES))
    lin_u, lin_i = _lin_kernel(ut, it)
    lin_u = lin_u.reshape(PACK_ROWS * 8 * TILE_W)
    lin_i = lin_i.reshape(PACK_ROWS * 8 * TILE_W)
    return _mf_kernel(uidx, iidx, lin_u, lin_i, tail_u, tail_i, w_b)


# speedup vs baseline: 18.4156x; 1.0291x over previous
import functools

import jax
import jax.numpy as jnp
from jax import lax
from jax.experimental import pallas as pl
from jax.experimental.pallas import tpu as pltpu
from jax.experimental.pallas import tpu_sc as plsc

NUM_CORES = 2
NUM_SUBCORES = 16
NUM_WORKERS = NUM_CORES * NUM_SUBCORES
LANES = 16

NUM_ROWS = 1000000
BATCH = 16384
DIM = 32
B_PER_W = BATCH // NUM_WORKERS
GROUPS = B_PER_W // LANES

TILE_W = 128
FULL_TILES = NUM_ROWS // TILE_W
ALIGNED = FULL_TILES * TILE_W
TAIL = NUM_ROWS - ALIGNED
DBLKS = DIM // 8
PACK_ROWS = DBLKS * FULL_TILES
TILES_PER_W = 246
LAST_START = FULL_TILES - TILES_PER_W
WAVE_T = 41
WAVES = TILES_PER_W // WAVE_T


def _lin_body(ut_hbm, it_hbm, lu_hbm, li_hbm, tbuf_a, tbuf_b, sem_r, sem_w):
    wid = lax.axis_index("s") * NUM_CORES + lax.axis_index("c")
    start_tile = jnp.minimum(wid * TILES_PER_W, LAST_START)

    for tab_hbm, lin_hbm in ((ut_hbm, lu_hbm), (it_hbm, li_hbm)):
        for dblk in range(DBLKS):

            def fire_reads(k, buf):
                cps = []
                for j in range(WAVE_T):
                    t = start_tile + k * WAVE_T + j
                    i0 = pl.multiple_of(t * TILE_W, TILE_W)
                    cps.append(pltpu.async_copy(
                        tab_hbm.at[pl.ds(dblk * 8, 8), pl.ds(i0, TILE_W)],
                        buf.at[j], sem_r))
                return cps

            def write_wave(k, buf):
                cps = []
                for j in range(WAVE_T):
                    t = start_tile + k * WAVE_T + j
                    cps.append(pltpu.async_copy(
                        buf.at[j], lin_hbm.at[dblk * FULL_TILES + t], sem_w))
                return cps

            def drain_writes():
                for j in range(WAVE_T):
                    pltpu.make_async_copy(
                        lin_hbm.at[0], tbuf_b.at[j], sem_w).wait()

            def wave_pair(p, carry):
                ra = fire_reads(2 * p, tbuf_a)

                @pl.when(p > 0)
                def _():
                    drain_writes()
                for c in ra:
                    c.wait()
                wa = write_wave(2 * p, tbuf_a)
                rb = fire_reads(2 * p + 1, tbuf_b)
                for c in rb:
                    c.wait()
                for c in wa:
                    c.wait()
                write_wave(2 * p + 1, tbuf_b)
                return carry

            lax.fori_loop(0, WAVES // 2, wave_pair, 0)
            drain_writes()


@functools.partial(
    pl.kernel,
    out_type=(jax.ShapeDtypeStruct((PACK_ROWS, 8, TILE_W), jnp.float32),
              jax.ShapeDtypeStruct((PACK_ROWS, 8, TILE_W), jnp.float32)),
    mesh=plsc.VectorSubcoreMesh(core_axis_name="c", subcore_axis_name="s"),
    scratch_types=[
        pltpu.VMEM((WAVE_T, 8, TILE_W), jnp.float32),
        pltpu.VMEM((WAVE_T, 8, TILE_W), jnp.float32),
        pltpu.SemaphoreType.DMA,
        pltpu.SemaphoreType.DMA,
    ],
    compiler_params=pltpu.CompilerParams(
        needs_layout_passes=False, use_tc_tiling_on_sc=True),
)
def _lin_kernel(*refs):
    _lin_body(*refs)


def _mf_body(uidx_hbm, iidx_hbm, lu_hbm, li_hbm, tu_hbm, ti_hbm, w_hbm,
             out_hbm, idx_u, idx_i, ids_u, ids_i, g_u, g_i, tl_u, tl_i,
             w_v, out_v, sem):
    wid = lax.axis_index("s") * NUM_CORES + lax.axis_index("c")
    base = wid * B_PER_W

    pltpu.sync_copy(uidx_hbm.at[wid], idx_u)
    pltpu.sync_copy(iidx_hbm.at[wid], idx_i)
    pltpu.sync_copy(w_hbm, w_v)
    pltpu.sync_copy(tu_hbm, tl_u)
    pltpu.sync_copy(ti_hbm, tl_i)

    clamp = jnp.full((LANES,), ALIGNED - 1, jnp.int32)
    m127 = jnp.full((LANES,), TILE_W - 1, jnp.int32)

    def build_body(g, carry):
        u16 = jnp.minimum(idx_u[pl.ds(g * LANES, LANES)], clamp)
        i16 = jnp.minimum(idx_i[pl.ds(g * LANES, LANES)], clamp)
        su = lax.shift_left(lax.shift_right_logical(u16, 7), 10) \
            + lax.bitwise_and(u16, m127)
        si = lax.shift_left(lax.shift_right_logical(i16, 7), 10) \
            + lax.bitwise_and(i16, m127)
        for d in range(DIM):
            cd = (d // 8) * FULL_TILES * 1024 + (d % 8) * TILE_W
            off = jnp.full((LANES,), cd, jnp.int32)
            ids_u[pl.ds(d * B_PER_W + g * LANES, LANES)] = su + off
            ids_i[pl.ds(d * B_PER_W + g * LANES, LANES)] = si + off
        return carry

    lax.fori_loop(0, GROUPS, build_body, 0)

    cu = pltpu.async_copy(lu_hbm.at[ids_u], g_u, sem)
    ci = pltpu.async_copy(li_hbm.at[ids_i], g_i, sem)
    cu.wait()
    ci.wait()

    tail_lo = jnp.full((LANES,), ALIGNED, jnp.int32)
    zero16 = jnp.zeros((LANES,), jnp.int32)

    def group_body(g, carry):
        iu16 = idx_u[pl.ds(g * LANES, LANES)]
        ii16 = idx_i[pl.ds(g * LANES, LANES)]
        mu = iu16 >= tail_lo
        mi = ii16 >= tail_lo
        tu16 = jnp.maximum(iu16 - tail_lo, zero16)
        ti16 = jnp.maximum(ii16 - tail_lo, zero16)
        acc = jnp.zeros((LANES,), jnp.float32)
        for d in range(DIM):
            off = d * B_PER_W + g * LANES
            d16 = jnp.full((LANES,), d, jnp.int32)
            uval = jnp.where(mu, plsc.load_gather(tl_u, [tu16, d16]),
                             g_u[pl.ds(off, LANES)])
            ival = jnp.where(mi, plsc.load_gather(tl_i, [ti16, d16]),
                             g_i[pl.ds(off, LANES)])
            acc = acc + uval * ival * w_v[d, :]
        sig = 1.0 / (1.0 + jnp.exp(-acc))
        out_v[pl.ds(g * LANES, LANES)] = sig
        return carry

    lax.fori_loop(0, GROUPS, group_body, 0)

    pltpu.sync_copy(out_v, out_hbm.at[pl.ds(base, B_PER_W)])


@functools.partial(
    pl.kernel,
    out_type=jax.ShapeDtypeStruct((BATCH,), jnp.float32),
    mesh=plsc.VectorSubcoreMesh(core_axis_name="c", subcore_axis_name="s"),
    scratch_types=[
        pltpu.VMEM((B_PER_W,), jnp.int32),
        pltpu.VMEM((B_PER_W,), jnp.int32),
        pltpu.VMEM((B_PER_W * DIM,), jnp.int32),
        pltpu.VMEM((B_PER_W * DIM,), jnp.int32),
        pltpu.VMEM((B_PER_W * DIM,), jnp.float32),
        pltpu.VMEM((B_PER_W * DIM,), jnp.float32),
        pltpu.VMEM((TAIL, DIM), jnp.float32),
        pltpu.VMEM((TAIL, DIM), jnp.float32),
        pltpu.VMEM((DIM, LANES), jnp.float32),
        pltpu.VMEM((B_PER_W,), jnp.float32),
        pltpu.SemaphoreType.DMA,
    ],
    compiler_params=pltpu.CompilerParams(
        needs_layout_passes=False, use_tc_tiling_on_sc=False),
)
def _mf_kernel(*refs):
    _mf_body(*refs)


def kernel(user_indices, item_indices, user_emb, item_emb, fc_w):
    uidx = user_indices.astype(jnp.int32).reshape(NUM_WORKERS, B_PER_W)
    iidx = item_indices.astype(jnp.int32).reshape(NUM_WORKERS, B_PER_W)
    ut = user_emb.T
    it = item_emb.T
    tail_u = user_emb[ALIGNED:, :]
    tail_i = item_emb[ALIGNED:, :]
    w_b = jnp.broadcast_to(fc_w.reshape(DIM, 1), (DIM, LANES))
    lin_u, lin_i = _lin_kernel(ut, it)
    lin_u = lin_u.reshape(PACK_ROWS * 8 * TILE_W)
    lin_i = lin_i.reshape(PACK_ROWS * 8 * TILE_W)
    return _mf_kernel(uidx, iidx, lin_u, lin_i, tail_u, tail_i, w_b)


# speedup vs baseline: 19.1149x; 1.0380x over previous
import functools

import jax
import jax.numpy as jnp
from jax import lax
from jax.experimental import pallas as pl
from jax.experimental.pallas import tpu as pltpu
from jax.experimental.pallas import tpu_sc as plsc

NUM_CORES = 2
NUM_SUBCORES = 16
NUM_WORKERS = NUM_CORES * NUM_SUBCORES
LANES = 16

NUM_ROWS = 1000000
BATCH = 16384
DIM = 32
B_PER_W = BATCH // NUM_WORKERS
GROUPS = B_PER_W // LANES

TILE_W = 128
FULL_TILES = NUM_ROWS // TILE_W
ALIGNED = FULL_TILES * TILE_W
TAIL = NUM_ROWS - ALIGNED
DBLKS = DIM // 8
PACK_ROWS = DBLKS * FULL_TILES
TILES_PER_W = 246
LAST_START = FULL_TILES - TILES_PER_W
WAVE_T = 41
WAVES = TILES_PER_W // WAVE_T


def _lin_body(ut_hbm, it_hbm, lu_hbm, li_hbm, tbuf_a, tbuf_b, sem_r, sem_w):
    wid = lax.axis_index("s") * NUM_CORES + lax.axis_index("c")
    start_tile = jnp.minimum(wid * TILES_PER_W, LAST_START)

    for tab_hbm, lin_hbm in ((ut_hbm, lu_hbm), (it_hbm, li_hbm)):

        def fire_reads(k, buf):
            dblk = k // WAVES
            kk = k % WAVES
            d0 = pl.multiple_of(dblk * 8, 8)
            cps = []
            for j in range(WAVE_T):
                t = start_tile + kk * WAVE_T + j
                i0 = pl.multiple_of(t * TILE_W, TILE_W)
                cps.append(pltpu.async_copy(
                    tab_hbm.at[pl.ds(d0, 8), pl.ds(i0, TILE_W)],
                    buf.at[j], sem_r))
            return cps

        def write_wave(k, buf):
            dblk = k // WAVES
            kk = k % WAVES
            cps = []
            for j in range(WAVE_T):
                t = start_tile + kk * WAVE_T + j
                cps.append(pltpu.async_copy(
                    buf.at[j], lin_hbm.at[dblk * FULL_TILES + t], sem_w))
            return cps

        def drain_writes():
            for j in range(WAVE_T):
                pltpu.make_async_copy(
                    lin_hbm.at[0], tbuf_b.at[j], sem_w).wait()

        def wave_pair(p, carry):
            ra = fire_reads(2 * p, tbuf_a)

            @pl.when(p > 0)
            def _():
                drain_writes()
            for c in ra:
                c.wait()
            wa = write_wave(2 * p, tbuf_a)
            rb = fire_reads(2 * p + 1, tbuf_b)
            for c in rb:
                c.wait()
            for c in wa:
                c.wait()
            write_wave(2 * p + 1, tbuf_b)
            return carry

        lax.fori_loop(0, DBLKS * WAVES // 2, wave_pair, 0)
        drain_writes()


@functools.partial(
    pl.kernel,
    out_type=(jax.ShapeDtypeStruct((PACK_ROWS, 8, TILE_W), jnp.float32),
              jax.ShapeDtypeStruct((PACK_ROWS, 8, TILE_W), jnp.float32)),
    mesh=plsc.VectorSubcoreMesh(core_axis_name="c", subcore_axis_name="s"),
    scratch_types=[
        pltpu.VMEM((WAVE_T, 8, TILE_W), jnp.float32),
        pltpu.VMEM((WAVE_T, 8, TILE_W), jnp.float32),
        pltpu.SemaphoreType.DMA,
        pltpu.SemaphoreType.DMA,
    ],
    compiler_params=pltpu.CompilerParams(
        needs_layout_passes=False, use_tc_tiling_on_sc=True),
)
def _lin_kernel(*refs):
    _lin_body(*refs)


def _mf_body(uidx_hbm, iidx_hbm, lu_hbm, li_hbm, tu_hbm, ti_hbm, w_hbm,
             out_hbm, idx_u, idx_i, ids_u, ids_i, g_u, g_i, tl_u, tl_i,
             w_v, out_v, sem):
    wid = lax.axis_index("s") * NUM_CORES + lax.axis_index("c")
    base = wid * B_PER_W

    pltpu.sync_copy(uidx_hbm.at[wid], idx_u)
    pltpu.sync_copy(iidx_hbm.at[wid], idx_i)
    pltpu.sync_copy(w_hbm, w_v)
    pltpu.sync_copy(tu_hbm, tl_u)
    pltpu.sync_copy(ti_hbm, tl_i)

    clamp = jnp.full((LANES,), ALIGNED - 1, jnp.int32)
    m127 = jnp.full((LANES,), TILE_W - 1, jnp.int32)

    def build_body(g, carry):
        u16 = jnp.minimum(idx_u[pl.ds(g * LANES, LANES)], clamp)
        i16 = jnp.minimum(idx_i[pl.ds(g * LANES, LANES)], clamp)
        su = lax.shift_left(lax.shift_right_logical(u16, 7), 10) \
            + lax.bitwise_and(u16, m127)
        si = lax.shift_left(lax.shift_right_logical(i16, 7), 10) \
            + lax.bitwise_and(i16, m127)
        for d in range(DIM):
            cd = (d // 8) * FULL_TILES * 1024 + (d % 8) * TILE_W
            off = jnp.full((LANES,), cd, jnp.int32)
            ids_u[pl.ds(d * B_PER_W + g * LANES, LANES)] = su + off
            ids_i[pl.ds(d * B_PER_W + g * LANES, LANES)] = si + off
        return carry

    lax.fori_loop(0, GROUPS, build_body, 0)

    cu = pltpu.async_copy(lu_hbm.at[ids_u], g_u, sem)
    ci = pltpu.async_copy(li_hbm.at[ids_i], g_i, sem)
    cu.wait()
    ci.wait()

    tail_lo = jnp.full((LANES,), ALIGNED, jnp.int32)
    zero16 = jnp.zeros((LANES,), jnp.int32)

    def group_body(g, carry):
        iu16 = idx_u[pl.ds(g * LANES, LANES)]
        ii16 = idx_i[pl.ds(g * LANES, LANES)]
        mu = iu16 >= tail_lo
        mi = ii16 >= tail_lo
        tu16 = jnp.maximum(iu16 - tail_lo, zero16)
        ti16 = jnp.maximum(ii16 - tail_lo, zero16)
        acc = jnp.zeros((LANES,), jnp.float32)
        for d in range(DIM):
            off = d * B_PER_W + g * LANES
            d16 = jnp.full((LANES,), d, jnp.int32)
            uval = jnp.where(mu, plsc.load_gather(tl_u, [tu16, d16]),
                             g_u[pl.ds(off, LANES)])
            ival = jnp.where(mi, plsc.load_gather(tl_i, [ti16, d16]),
                             g_i[pl.ds(off, LANES)])
            acc = acc + uval * ival * w_v[d, :]
        sig = 1.0 / (1.0 + jnp.exp(-acc))
        out_v[pl.ds(g * LANES, LANES)] = sig
        return carry

    lax.fori_loop(0, GROUPS, group_body, 0)

    pltpu.sync_copy(out_v, out_hbm.at[pl.ds(base, B_PER_W)])


@functools.partial(
    pl.kernel,
    out_type=jax.ShapeDtypeStruct((BATCH,), jnp.float32),
    mesh=plsc.VectorSubcoreMesh(core_axis_name="c", subcore_axis_name="s"),
    scratch_types=[
        pltpu.VMEM((B_PER_W,), jnp.int32),
        pltpu.VMEM((B_PER_W,), jnp.int32),
        pltpu.VMEM((B_PER_W * DIM,), jnp.int32),
        pltpu.VMEM((B_PER_W * DIM,), jnp.int32),
        pltpu.VMEM((B_PER_W * DIM,), jnp.float32),
        pltpu.VMEM((B_PER_W * DIM,), jnp.float32),
        pltpu.VMEM((TAIL, DIM), jnp.float32),
        pltpu.VMEM((TAIL, DIM), jnp.float32),
        pltpu.VMEM((DIM, LANES), jnp.float32),
        pltpu.VMEM((B_PER_W,), jnp.float32),
        pltpu.SemaphoreType.DMA,
    ],
    compiler_params=pltpu.CompilerParams(
        needs_layout_passes=False, use_tc_tiling_on_sc=False),
)
def _mf_kernel(*refs):
    _mf_body(*refs)


def kernel(user_indices, item_indices, user_emb, item_emb, fc_w):
    uidx = user_indices.astype(jnp.int32).reshape(NUM_WORKERS, B_PER_W)
    iidx = item_indices.astype(jnp.int32).reshape(NUM_WORKERS, B_PER_W)
    ut = user_emb.T
    it = item_emb.T
    tail_u = user_emb[ALIGNED:, :]
    tail_i = item_emb[ALIGNED:, :]
    w_b = jnp.broadcast_to(fc_w.reshape(DIM, 1), (DIM, LANES))
    lin_u, lin_i = _lin_kernel(ut, it)
    lin_u = lin_u.reshape(PACK_ROWS * 8 * TILE_W)
    lin_i = lin_i.reshape(PACK_ROWS * 8 * TILE_W)
    return _mf_kernel(uidx, iidx, lin_u, lin_i, tail_u, tail_i, w_b)
